# Initial kernel scaffold; baseline (speedup 1.0000x reference)
import jax
import jax.numpy as jnp
from jax.experimental import pallas as pl


def kernel(x, edge, WQ, bQ, WK, bK, WV, bV):
    raise NotImplementedError("write your pallas kernel here")


# speedup vs baseline: 3.5698x; 3.5698x over previous
import functools
import math

import jax
import jax.numpy as jnp
from jax import lax
from jax.experimental import pallas as pl
from jax.experimental.pallas import tpu as pltpu
from jax.experimental.pallas import tpu_sc as plsc

N = 10000
E = 320000
F = 128
H = 4
DK = 32
L = 16
NC = 2
NS = 16
NW = NC * NS
EPT = E // NW
C1 = 80
NCH1 = EPT // C1
C2 = 2000
NCH2 = EPT // C2
INV_SQRT_DK = 1.0 / math.sqrt(DK)

BR = 400


def _proj_body(x_ref, wqt, wkt, wvt, bq, bk, bv, q_ref, k_ref, v_ref):
    xb = x_ref[...]
    q_ref[...] = jnp.dot(xb, wqt[...], preferred_element_type=jnp.float32) + bq[...]
    k_ref[...] = jnp.dot(xb, wkt[...], preferred_element_type=jnp.float32) + bk[...]
    v_ref[...] = jnp.dot(xb, wvt[...], preferred_element_type=jnp.float32) + bv[...]


_project = pl.pallas_call(
    _proj_body,
    grid=(N // BR,),
    in_specs=[
        pl.BlockSpec((BR, F), lambda i: (i, 0)),
        pl.BlockSpec((F, F), lambda i: (0, 0)),
        pl.BlockSpec((F, F), lambda i: (0, 0)),
        pl.BlockSpec((F, F), lambda i: (0, 0)),
        pl.BlockSpec((1, F), lambda i: (0, 0)),
        pl.BlockSpec((1, F), lambda i: (0, 0)),
        pl.BlockSpec((1, F), lambda i: (0, 0)),
    ],
    out_specs=[pl.BlockSpec((BR, F), lambda i: (i, 0))] * 3,
    out_shape=[jax.ShapeDtypeStruct((N, F), jnp.float32)] * 3,
)

_sc_params = pltpu.CompilerParams(
    needs_layout_passes=False, use_tc_tiling_on_sc=False
)


@functools.partial(
    pl.kernel,
    out_type=[
        jax.ShapeDtypeStruct((H, E), jnp.float32),
        jax.ShapeDtypeStruct((H, E), jnp.float32),
        jax.ShapeDtypeStruct((H, N), jnp.float32),
        jax.ShapeDtypeStruct((H, N), jnp.float32),
    ],
    mesh=plsc.VectorSubcoreMesh(core_axis_name="c", subcore_axis_name="s"),
    scratch_types=[
        pltpu.VMEM((C1,), jnp.int32),
        pltpu.VMEM((C1,), jnp.int32),
        pltpu.VMEM((C1, F), jnp.float32),
        pltpu.VMEM((C1, F), jnp.float32),
        [pltpu.VMEM((C1,), jnp.float32)] * H,
        [pltpu.VMEM((C1,), jnp.float32)] * H,
        [pltpu.VMEM_SHARED((N,), jnp.float32)] * H,
        pltpu.VMEM((N,), jnp.float32),
        pltpu.SemaphoreType.DMA,
        pltpu.SemaphoreType.DMA,
    ],
    compiler_params=_sc_params,
)
def _pass1(qtab, ktab, e0, e1,
           prods_hbm, expv_hbm, den0_hbm, den1_hbm,
           idx0, idx1, qrows, krows, pch, ech, den_sh, stage, sem_q, sem_k):
    cid = lax.axis_index("c")
    sid = lax.axis_index("s")
    wid = cid * NS + sid
    base = wid * EPT

    @pl.when(sid == 0)
    def _():
        zv = jnp.zeros((L,), jnp.float32)

        def zbody(i, carry):
            stage[pl.ds(i * L, L)] = zv
            return carry

        lax.fori_loop(0, N // L, zbody, 0)
        for h in range(H):
            pltpu.sync_copy(stage, den_sh[h])

    plsc.subcore_barrier()

    lane = lax.iota(jnp.int32, L)

    def chunk_body(c, carry):
        off = base + c * C1
        pltpu.sync_copy(e0.at[pl.ds(off, C1)], idx0)
        pltpu.sync_copy(e1.at[pl.ds(off, C1)], idx1)
        cq = pltpu.async_copy(qtab.at[idx0], qrows, sem_q)
        ck = pltpu.async_copy(ktab.at[idx1], krows, sem_k)
        cq.wait()
        ck.wait()

        def edge_body(eb, carry2):
            rows = eb * L + lane
            for h in range(H):
                acc = jnp.zeros((L,), jnp.float32)
                for f in range(DK):
                    col = jnp.full((L,), h * DK + f, jnp.int32)
                    qv = plsc.load_gather(qrows, [rows, col])
                    kv = plsc.load_gather(krows, [rows, col])
                    acc = acc + qv * kv
                acc = acc * INV_SQRT_DK
                pch[h][pl.ds(eb * L, L)] = acc
                ech[h][pl.ds(eb * L, L)] = jnp.exp(acc)
            return carry2

        lax.fori_loop(0, C1 // L, edge_body, 0)

        for h in range(H):
            pltpu.sync_copy(pch[h], prods_hbm.at[h, pl.ds(off, C1)])
            pltpu.sync_copy(ech[h], expv_hbm.at[h, pl.ds(off, C1)])
            pltpu.sync_copy(ech[h], den_sh[h].at[idx0], add=True)
        return carry

    lax.fori_loop(0, NCH1, chunk_body, 0)

    plsc.subcore_barrier()

    @pl.when((sid == 0) & (cid == 0))
    def _():
        for h in range(H):
            pltpu.sync_copy(den_sh[h], stage)
            pltpu.sync_copy(stage, den0_hbm.at[h])

    @pl.when((sid == 0) & (cid == 1))
    def _():
        for h in range(H):
            pltpu.sync_copy(den_sh[h], stage)
            pltpu.sync_copy(stage, den1_hbm.at[h])


@functools.partial(
    pl.kernel,
    out_type=jax.ShapeDtypeStruct((H, E), jnp.float32),
    mesh=plsc.VectorSubcoreMesh(core_axis_name="c", subcore_axis_name="s"),
    scratch_types=[
        [pltpu.VMEM((N,), jnp.float32)] * H,
        [pltpu.VMEM((N,), jnp.float32)] * H,
        pltpu.VMEM((C2,), jnp.int32),
        [pltpu.VMEM((C2,), jnp.float32)] * H,
        [pltpu.VMEM((C2,), jnp.float32)] * H,
    ],
    compiler_params=_sc_params,
)
def _pass2(e0, expv_hbm, den0_hbm, den1_hbm, att_hbm,
           den0v, den1v, idx0, ech, ach):
    cid = lax.axis_index("c")
    sid = lax.axis_index("s")
    wid = cid * NS + sid
    base = wid * EPT
    for h in range(H):
        pltpu.sync_copy(den0_hbm.at[h], den0v[h])
        pltpu.sync_copy(den1_hbm.at[h], den1v[h])

    def chunk_body(c, carry):
        off = base + c * C2
        pltpu.sync_copy(e0.at[pl.ds(off, C2)], idx0)
        for h in range(H):
            pltpu.sync_copy(expv_hbm.at[h, pl.ds(off, C2)], ech[h])

        def blk(i, carry2):
            seg = idx0[pl.ds(i * L, L)]
            for h in range(H):
                d0 = plsc.load_gather(den0v[h], [seg])
                d1 = plsc.load_gather(den1v[h], [seg])
                ev = ech[h][pl.ds(i * L, L)]
                ach[h][pl.ds(i * L, L)] = ev / (d0 + d1 + 1e-16)
            return carry2

        lax.fori_loop(0, C2 // L, blk, 0)
        for h in range(H):
            pltpu.sync_copy(ach[h], att_hbm.at[h, pl.ds(off, C2)])
        return carry

    lax.fori_loop(0, NCH2, chunk_body, 0)


def kernel(x, edge, WQ, bQ, WK, bK, WV, bV):
    q, k, v = _project(x, WQ.T, WK.T, WV.T,
                       bQ.reshape(1, F), bK.reshape(1, F), bV.reshape(1, F))
    e0 = edge[0]
    e1 = edge[1]
    prods_hm, expv_hm, den0, den1 = _pass1(q, k, e0, e1)
    att_hm = _pass2(e0, expv_hm, den0, den1)
    v3 = v.reshape(N, H, DK).transpose(0, 2, 1)
    return (att_hm.T, v3, prods_hm.T)


# speedup vs baseline: 4.0108x; 1.1235x over previous
import functools
import math

import jax
import jax.numpy as jnp
from jax import lax
from jax.experimental import pallas as pl
from jax.experimental.pallas import tpu as pltpu
from jax.experimental.pallas import tpu_sc as plsc

N = 10000
E = 320000
F = 128
H = 4
DK = 32
L = 16
NC = 2
NS = 16
NW = NC * NS
EPT = E // NW
C1 = 400
NCH1 = EPT // C1
NCHT = E // C1
CB2 = 5
NCH2 = NCH1 // CB2
INV_SQRT_DK = 1.0 / math.sqrt(DK)

BR = 400


def _proj_body(x_ref, wqt, wkt, wvt, bq, bk, bv, q_ref, k_ref, v_ref):
    xb = x_ref[...]
    q_ref[...] = jnp.dot(xb, wqt[...], preferred_element_type=jnp.float32) + bq[...]
    k_ref[...] = jnp.dot(xb, wkt[...], preferred_element_type=jnp.float32) + bk[...]
    v_ref[...] = jnp.dot(xb, wvt[...], preferred_element_type=jnp.float32) + bv[...]


_project = pl.pallas_call(
    _proj_body,
    grid=(N // BR,),
    in_specs=[
        pl.BlockSpec((BR, F), lambda i: (i, 0)),
        pl.BlockSpec((F, F), lambda i: (0, 0)),
        pl.BlockSpec((F, F), lambda i: (0, 0)),
        pl.BlockSpec((F, F), lambda i: (0, 0)),
        pl.BlockSpec((1, F), lambda i: (0, 0)),
        pl.BlockSpec((1, F), lambda i: (0, 0)),
        pl.BlockSpec((1, F), lambda i: (0, 0)),
    ],
    out_specs=[pl.BlockSpec((BR, F), lambda i: (i, 0))] * 3,
    out_shape=[jax.ShapeDtypeStruct((N, F), jnp.float32)] * 3,
)

_sc_params = pltpu.CompilerParams(
    needs_layout_passes=False, use_tc_tiling_on_sc=False
)


@functools.partial(
    pl.kernel,
    out_type=[
        jax.ShapeDtypeStruct((NCHT, H, C1), jnp.float32),
        jax.ShapeDtypeStruct((NCHT, H, C1), jnp.float32),
        jax.ShapeDtypeStruct((H, N), jnp.float32),
        jax.ShapeDtypeStruct((H, N), jnp.float32),
    ],
    mesh=plsc.VectorSubcoreMesh(core_axis_name="c", subcore_axis_name="s"),
    scratch_types=[
        pltpu.VMEM((C1,), jnp.int32),
        pltpu.VMEM((C1,), jnp.int32),
        pltpu.VMEM((C1, F), jnp.float32),
        pltpu.VMEM((C1, F), jnp.float32),
        pltpu.VMEM((H, C1), jnp.float32),
        pltpu.VMEM((H, C1), jnp.float32),
        [pltpu.VMEM_SHARED((N,), jnp.float32)] * H,
        pltpu.VMEM((N,), jnp.float32),
        pltpu.SemaphoreType.DMA,
        pltpu.SemaphoreType.DMA,
    ],
    compiler_params=_sc_params,
)
def _pass1(qtab, ktab, e0, e1,
           prods_hbm, expv_hbm, den0_hbm, den1_hbm,
           idx0, idx1, qrows, krows, pch, ech, den_sh, stage, sem_q, sem_k):
    cid = lax.axis_index("c")
    sid = lax.axis_index("s")
    wid = cid * NS + sid
    base = wid * EPT

    @pl.when(sid == 0)
    def _():
        zv = jnp.zeros((L,), jnp.float32)

        def zbody(i, carry):
            stage[pl.ds(i * L, L)] = zv
            return carry

        lax.fori_loop(0, N // L, zbody, 0)
        for h in range(H):
            pltpu.sync_copy(stage, den_sh[h])

    plsc.subcore_barrier()

    lane = lax.iota(jnp.int32, L)

    def chunk_body(c, carry):
        off = base + c * C1
        g = wid * NCH1 + c
        pltpu.sync_copy(e0.at[pl.ds(off, C1)], idx0)
        pltpu.sync_copy(e1.at[pl.ds(off, C1)], idx1)
        cq = pltpu.async_copy(qtab.at[idx0], qrows, sem_q)
        ck = pltpu.async_copy(ktab.at[idx1], krows, sem_k)
        cq.wait()
        ck.wait()

        def edge_body(eb, carry2):
            rows = eb * L + lane
            for h in range(H):
                acc = jnp.zeros((L,), jnp.float32)
                for f in range(DK):
                    col = jnp.full((L,), h * DK + f, jnp.int32)
                    qv = plsc.load_gather(qrows, [rows, col])
                    kv = plsc.load_gather(krows, [rows, col])
                    acc = acc + qv * kv
                acc = acc * INV_SQRT_DK
                pch[h, pl.ds(eb * L, L)] = acc
                ech[h, pl.ds(eb * L, L)] = jnp.exp(acc)
            return carry2

        lax.fori_loop(0, C1 // L, edge_body, 0)

        pltpu.sync_copy(pch, prods_hbm.at[g])
        pltpu.sync_copy(ech, expv_hbm.at[g])
        for h in range(H):
            pltpu.sync_copy(ech.at[h], den_sh[h].at[idx0], add=True)
        return carry

    lax.fori_loop(0, NCH1, chunk_body, 0)

    plsc.subcore_barrier()

    @pl.when((sid == 0) & (cid == 0))
    def _():
        for h in range(H):
            pltpu.sync_copy(den_sh[h], stage)
            pltpu.sync_copy(stage, den0_hbm.at[h])

    @pl.when((sid == 0) & (cid == 1))
    def _():
        for h in range(H):
            pltpu.sync_copy(den_sh[h], stage)
            pltpu.sync_copy(stage, den1_hbm.at[h])


@functools.partial(
    pl.kernel,
    out_type=jax.ShapeDtypeStruct((NCHT, H, C1), jnp.float32),
    mesh=plsc.VectorSubcoreMesh(core_axis_name="c", subcore_axis_name="s"),
    scratch_types=[
        [pltpu.VMEM((N,), jnp.float32)] * H,
        [pltpu.VMEM((N,), jnp.float32)] * H,
        pltpu.VMEM((CB2 * C1,), jnp.int32),
        pltpu.VMEM((CB2, H, C1), jnp.float32),
        pltpu.VMEM((CB2, H, C1), jnp.float32),
    ],
    compiler_params=_sc_params,
)
def _pass2(e0, expv_hbm, den0_hbm, den1_hbm, att_hbm,
           den0v, den1v, idx0, ech, ach):
    cid = lax.axis_index("c")
    sid = lax.axis_index("s")
    wid = cid * NS + sid
    base = wid * EPT
    for h in range(H):
        pltpu.sync_copy(den0_hbm.at[h], den0v[h])
        pltpu.sync_copy(den1_hbm.at[h], den1v[h])

    def chunk_body(c, carry):
        off = base + c * (CB2 * C1)
        g0 = wid * NCH1 + c * CB2
        pltpu.sync_copy(e0.at[pl.ds(off, CB2 * C1)], idx0)
        pltpu.sync_copy(expv_hbm.at[pl.ds(g0, CB2)], ech)

        def jbody(j, carry2):
            def blk(r, carry3):
                seg = idx0[pl.ds(j * C1 + r * L, L)]
                for h in range(H):
                    d0 = plsc.load_gather(den0v[h], [seg])
                    d1 = plsc.load_gather(den1v[h], [seg])
                    ev = ech[j, h, pl.ds(r * L, L)]
                    ach[j, h, pl.ds(r * L, L)] = ev / (d0 + d1 + 1e-16)
                return carry3

            lax.fori_loop(0, C1 // L, blk, 0)
            return carry2

        lax.fori_loop(0, CB2, jbody, 0)
        pltpu.sync_copy(ach, att_hbm.at[pl.ds(g0, CB2)])
        return carry

    lax.fori_loop(0, NCH2, chunk_body, 0)


def kernel(x, edge, WQ, bQ, WK, bK, WV, bV):
    q, k, v = _project(x, WQ.T, WK.T, WV.T,
                       bQ.reshape(1, F), bK.reshape(1, F), bV.reshape(1, F))
    e0 = edge[0]
    e1 = edge[1]
    prods_b, expv_b, den0, den1 = _pass1(q, k, e0, e1)
    att_b = _pass2(e0, expv_b, den0, den1)
    v3 = v.reshape(N, H, DK).transpose(0, 2, 1)
    att = att_b.transpose(0, 2, 1).reshape(E, H)
    prods = prods_b.transpose(0, 2, 1).reshape(E, H)
    return (att, v3, prods)
